# baseline (speedup 1.0000x reference)
import functools

import jax
import jax.numpy as jnp
from jax import lax
from jax.experimental import pallas as pl
from jax.experimental.pallas import tpu as pltpu
from jax.experimental.pallas import tpu_sc as plsc

B = 16384
D = 64
LANES = 16
PAD = 17
PW = 2 * D

_info = plsc.get_sparse_core_info()
NC = _info.num_cores
NS = _info.num_subcores
NW = NC * NS
BPW = B // NW
GCHUNK = 128
NG = BPW // GCHUNK

_mesh = plsc.VectorSubcoreMesh(core_axis_name="c", subcore_axis_name="s")


@functools.partial(
    pl.kernel,
    out_type=jax.ShapeDtypeStruct((B,), jnp.float32),
    mesh=_mesh,
    compiler_params=pltpu.CompilerParams(needs_layout_passes=False),
    scratch_types=[
        pltpu.VMEM((BPW,), jnp.int32),
        pltpu.VMEM((BPW,), jnp.int32),
        pltpu.VMEM((BPW,), jnp.int32),
        pltpu.VMEM((BPW,), jnp.int32),
        pltpu.VMEM((GCHUNK, PW), jnp.float32),
        pltpu.VMEM((GCHUNK, PW), jnp.float32),
        pltpu.VMEM((BPW * PAD,), jnp.float32),
        pltpu.VMEM((BPW,), jnp.float32),
        pltpu.SemaphoreType.DMA,
        pltpu.SemaphoreType.DMA,
    ],
)
def _sc_dot(user_hbm, item_hbm, uemb_hbm, iemb_hbm, out_hbm,
            uidx, iidx, uhi, ihi, utile, itile, part, outc, usem, isem):
    wid = lax.axis_index("s") * NC + lax.axis_index("c")
    base = wid * BPW

    pltpu.sync_copy(user_hbm.at[pl.ds(base, BPW)], uidx)
    pltpu.sync_copy(item_hbm.at[pl.ds(base, BPW)], iidx)

    def hi_body(k, carry):
        uhi[pl.ds(k * LANES, LANES)] = uidx[pl.ds(k * LANES, LANES)] >> 1
        ihi[pl.ds(k * LANES, LANES)] = iidx[pl.ds(k * LANES, LANES)] >> 1
        return carry

    lax.fori_loop(0, BPW // LANES, hi_body, 0, unroll=2)

    def chunk_body(g, carry):
        cu = pltpu.async_copy(
            uemb_hbm.at[uhi.at[pl.ds(g * GCHUNK, GCHUNK)]], utile, usem)
        ci = pltpu.async_copy(
            iemb_hbm.at[ihi.at[pl.ds(g * GCHUNK, GCHUNK)]], itile, isem)
        cu.wait()
        ci.wait()

        def ext_body(k2, carry2):
            e0 = k2 * LANES
            lu_vec = (uidx[pl.ds(g * GCHUNK + e0, LANES)] & 1) << 6
            li_vec = (iidx[pl.ds(g * GCHUNK + e0, LANES)] & 1) << 6
            for j in range(LANES):
                e2 = e0 + j
                lu = lu_vec[j]
                li = li_vec[j]
                acc = (utile[e2, pl.ds(lu, LANES)]
                       * itile[e2, pl.ds(li, LANES)])
                for k in range(1, D // LANES):
                    acc += (utile[e2, pl.ds(lu + k * LANES, LANES)]
                            * itile[e2, pl.ds(li + k * LANES, LANES)])
                part[pl.ds((g * GCHUNK + e2) * PAD, LANES)] = acc
            return carry2

        lax.fori_loop(0, GCHUNK // LANES, ext_body, 0)
        return carry

    lax.fori_loop(0, NG, chunk_body, 0)

    def grp_body(g, carry):
        rows = (g * LANES + lax.iota(jnp.int32, LANES)) * PAD
        acc = plsc.load_gather(part, [rows])
        for j in range(1, LANES):
            acc += plsc.load_gather(part, [rows + j])
        outc[pl.ds(g * LANES, LANES)] = acc
        return carry

    lax.fori_loop(0, BPW // LANES, grp_body, 0, unroll=2)

    pltpu.sync_copy(outc, out_hbm.at[pl.ds(base, BPW)])


def kernel(user, item, attr, user_embedding, item_embedding):
    del attr
    uemb = user_embedding.reshape(1000000 // 2, PW)
    iemb = item_embedding.reshape(1000000 // 2, PW)
    return _sc_dot(user.astype(jnp.int32), item.astype(jnp.int32), uemb, iemb)


# speedup vs baseline: 1.5673x; 1.5673x over previous
import functools

import jax
import jax.numpy as jnp
from jax import lax
from jax.experimental import pallas as pl
from jax.experimental.pallas import tpu as pltpu
from jax.experimental.pallas import tpu_sc as plsc

B = 16384
D = 64
LANES = 16
PAD = 17

_info = plsc.get_sparse_core_info()
NC = _info.num_cores
NS = _info.num_subcores
NW = NC * NS
BPW = B // NW
CH = 32
NCH = BPW // CH

_mesh = plsc.VectorSubcoreMesh(core_axis_name="c", subcore_axis_name="s")


@functools.partial(
    pl.kernel,
    out_type=jax.ShapeDtypeStruct((B,), jnp.float32),
    mesh=_mesh,
    compiler_params=pltpu.CompilerParams(needs_layout_passes=False),
    scratch_types=[
        pltpu.VMEM((BPW,), jnp.int32),
        pltpu.VMEM((BPW,), jnp.int32),
        pltpu.VMEM((CH, D), jnp.float32),
        pltpu.VMEM((CH, D), jnp.float32),
        pltpu.VMEM((BPW * PAD,), jnp.float32),
        pltpu.VMEM((BPW,), jnp.float32),
        pltpu.SemaphoreType.DMA,
        pltpu.SemaphoreType.DMA,
    ],
)
def _sc_dot(user_hbm, item_hbm, uemb_hbm, iemb_hbm, out_hbm,
            uidx, iidx, urows, irows, part, outc, usem, isem):
    wid = lax.axis_index("s") * NC + lax.axis_index("c")
    base = wid * BPW

    pltpu.sync_copy(user_hbm.at[pl.ds(base, BPW)], uidx)
    pltpu.sync_copy(item_hbm.at[pl.ds(base, BPW)], iidx)

    def chunk_body(g, carry):
        descs = []
        for k2 in range(CH // LANES):
            uvec = uidx[pl.ds(g * CH + k2 * LANES, LANES)]
            ivec = iidx[pl.ds(g * CH + k2 * LANES, LANES)]
            for j in range(LANES):
                e2 = k2 * LANES + j
                descs.append(pltpu.async_copy(
                    uemb_hbm.at[pl.ds(uvec[j], 1)],
                    urows.at[pl.ds(e2, 1)], usem))
                descs.append(pltpu.async_copy(
                    iemb_hbm.at[pl.ds(ivec[j], 1)],
                    irows.at[pl.ds(e2, 1)], isem))
        for dsc in descs:
            dsc.wait()

        def row_body(r, carry2):
            acc = urows[r, pl.ds(0, LANES)] * irows[r, pl.ds(0, LANES)]
            for k in range(1, D // LANES):
                acc += (urows[r, pl.ds(k * LANES, LANES)]
                        * irows[r, pl.ds(k * LANES, LANES)])
            part[pl.ds((g * CH + r) * PAD, LANES)] = acc
            return carry2

        lax.fori_loop(0, CH, row_body, 0, unroll=2)
        return carry

    lax.fori_loop(0, NCH, chunk_body, 0)

    def grp_body(g, carry):
        rows = (g * LANES + lax.iota(jnp.int32, LANES)) * PAD
        acc = plsc.load_gather(part, [rows])
        for j in range(1, LANES):
            acc += plsc.load_gather(part, [rows + j])
        outc[pl.ds(g * LANES, LANES)] = acc
        return carry

    lax.fori_loop(0, BPW // LANES, grp_body, 0, unroll=2)

    pltpu.sync_copy(outc, out_hbm.at[pl.ds(base, BPW)])


def kernel(user, item, attr, user_embedding, item_embedding):
    del attr
    return _sc_dot(user.astype(jnp.int32), item.astype(jnp.int32),
                   user_embedding, item_embedding)


# speedup vs baseline: 1.5707x; 1.0022x over previous
import functools

import jax
import jax.numpy as jnp
from jax import lax
from jax.experimental import pallas as pl
from jax.experimental.pallas import tpu as pltpu
from jax.experimental.pallas import tpu_sc as plsc

B = 16384
D = 64
LANES = 16
PAD = 17

_info = plsc.get_sparse_core_info()
NC = _info.num_cores
NS = _info.num_subcores
NW = NC * NS
BPW = B // NW
CH = 32
NCH = BPW // CH

_mesh = plsc.VectorSubcoreMesh(core_axis_name="c", subcore_axis_name="s")


@functools.partial(
    pl.kernel,
    out_type=jax.ShapeDtypeStruct((B,), jnp.float32),
    mesh=_mesh,
    compiler_params=pltpu.CompilerParams(
        needs_layout_passes=False, use_tc_tiling_on_sc=True),
    scratch_types=[
        pltpu.VMEM((BPW,), jnp.int32),
        pltpu.VMEM((BPW,), jnp.int32),
        pltpu.VMEM((CH, D), jnp.float32),
        pltpu.VMEM((CH, D), jnp.float32),
        pltpu.VMEM((BPW * PAD,), jnp.float32),
        pltpu.VMEM((BPW,), jnp.float32),
        pltpu.SemaphoreType.DMA,
        pltpu.SemaphoreType.DMA,
    ],
)
def _sc_dot(user_hbm, item_hbm, uemb_hbm, iemb_hbm, out_hbm,
            uidx, iidx, urows, irows, part, outc, usem, isem):
    wid = lax.axis_index("s") * NC + lax.axis_index("c")
    base = wid * BPW

    pltpu.sync_copy(user_hbm.at[pl.ds(base, BPW)], uidx)
    pltpu.sync_copy(item_hbm.at[pl.ds(base, BPW)], iidx)

    def chunk_body(g, carry):
        descs = []
        for k2 in range(CH // LANES):
            uvec = uidx[pl.ds(g * CH + k2 * LANES, LANES)]
            ivec = iidx[pl.ds(g * CH + k2 * LANES, LANES)]
            for j in range(LANES):
                e2 = k2 * LANES + j
                descs.append(pltpu.async_copy(
                    uemb_hbm.at[pl.ds(uvec[j], 1)],
                    urows.at[pl.ds(e2, 1)], usem))
                descs.append(pltpu.async_copy(
                    iemb_hbm.at[pl.ds(ivec[j], 1)],
                    irows.at[pl.ds(e2, 1)], isem))
        for dsc in descs:
            dsc.wait()

        def row_body(r, carry2):
            acc = urows[r, pl.ds(0, LANES)] * irows[r, pl.ds(0, LANES)]
            for k in range(1, D // LANES):
                acc += (urows[r, pl.ds(k * LANES, LANES)]
                        * irows[r, pl.ds(k * LANES, LANES)])
            part[pl.ds((g * CH + r) * PAD, LANES)] = acc
            return carry2

        lax.fori_loop(0, CH, row_body, 0, unroll=2)
        return carry

    lax.fori_loop(0, NCH, chunk_body, 0)

    def grp_body(g, carry):
        rows = (g * LANES + lax.iota(jnp.int32, LANES)) * PAD
        acc = plsc.load_gather(part, [rows])
        for j in range(1, LANES):
            acc += plsc.load_gather(part, [rows + j])
        outc[pl.ds(g * LANES, LANES)] = acc
        return carry

    lax.fori_loop(0, BPW // LANES, grp_body, 0, unroll=2)

    pltpu.sync_copy(outc, out_hbm.at[pl.ds(base, BPW)])


def kernel(user, item, attr, user_embedding, item_embedding):
    del attr
    return _sc_dot(user.astype(jnp.int32), item.astype(jnp.int32),
                   user_embedding, item_embedding)
